# baseline (speedup 1.0000x reference)
import jax
import jax.numpy as jnp
from jax import lax
from jax.experimental import pallas as pl
from jax.experimental.pallas import tpu as pltpu

_K = 8192
_C = 64
_BETA = 0.25
_B, _H, _W = 8, 32, 32
_HW = _H * _W
_HH = _HW // 2
_KC = 512
_NCH = _K // _KC


def _vq_body(z_ref, e_ref, et_ref, ids_ref, zq_ref, loss_ref,
             s0_ref, s1_ref, zn_ref, acc_ref):
    b = pl.program_id(0)
    z = z_ref[0]
    nsq = jnp.sum(z * z, axis=0, keepdims=True)
    n = jnp.sqrt(nsq)
    zn = z / jnp.maximum(n, 1e-12)
    zn_ref[...] = zn
    zn_sq = jnp.sum(zn * zn)

    def pass_a(k, rmax0):
        e_blk = e_ref[pl.ds(k * _KC, _KC), :]
        sim = lax.dot(e_blk, zn_ref[:, :_HH])
        s0_ref[k] = sim
        return jnp.maximum(rmax0, jnp.max(sim, axis=0, keepdims=True))

    rmax0 = lax.fori_loop(
        0, _NCH, pass_a, jnp.full((1, _HH), -jnp.inf, dtype=jnp.float32))

    acc_ref[...] = jnp.zeros((_C + 3, _HW), jnp.float32)

    def pass_b(k, rmax1):
        e_blk = e_ref[pl.ds(k * _KC, _KC), :]
        sim = lax.dot(e_blk, zn_ref[:, _HH:])
        s1_ref[k] = sim
        rmax1 = jnp.maximum(rmax1, jnp.max(sim, axis=0, keepdims=True))
        onehot = (s0_ref[k] == rmax0).astype(jnp.bfloat16)
        g_blk = et_ref[:, pl.ds(k * _KC, _KC)]
        acc_ref[:, :_HH] += lax.dot(g_blk, onehot,
                                    preferred_element_type=jnp.float32)
        return rmax1

    rmax1 = lax.fori_loop(
        0, _NCH, pass_b, jnp.full((1, _HH), -jnp.inf, dtype=jnp.float32))

    def pass_c(k, _):
        onehot = (s1_ref[k] == rmax1).astype(jnp.bfloat16)
        g_blk = et_ref[:, pl.ds(k * _KC, _KC)]
        acc_ref[:, _HH:] += lax.dot(g_blk, onehot,
                                    preferred_element_type=jnp.float32)
        return 0

    lax.fori_loop(0, _NCH, pass_c, 0)

    acc = acc_ref[...]
    count = acc[_C + 2:_C + 3]
    ids = (acc[_C:_C + 1] * 128.0 + acc[_C + 1:_C + 2]).astype(jnp.int32)
    ids_ref[0] = ids
    zq_ref[0] = acc[:_C]

    has_tie = jnp.any(count != 1.0)

    @pl.when(has_tie)
    def _():
        rmax = jnp.concatenate([rmax0, rmax1], axis=1)

        def find(k, ridx):
            iota = lax.broadcasted_iota(jnp.int32, (_KC, _HW), 0) + k * _KC
            sim = jnp.concatenate([s0_ref[k], s1_ref[k]], axis=1)
            bidx = jnp.min(jnp.where(sim == rmax, iota, _K),
                           axis=0, keepdims=True)
            return jnp.minimum(ridx, bidx)

        ids_x = lax.fori_loop(0, _NCH, find,
                              jnp.full((1, _HW), _K, dtype=jnp.int32))
        ids_ref[0] = ids_x
        acc_ref[...] = jnp.zeros((_C + 3, _HW), jnp.float32)

        def rebuild(k, _):
            iota = lax.broadcasted_iota(jnp.int32, (_KC, _HW), 0) + k * _KC
            onehot = (iota == ids_x).astype(jnp.bfloat16)
            g_blk = et_ref[:, pl.ds(k * _KC, _KC)]
            acc_ref[...] += lax.dot(g_blk, onehot,
                                    preferred_element_type=jnp.float32)
            return 0

        lax.fori_loop(0, _NCH, rebuild, 0)
        zq_ref[0] = acc_ref[:_C]

    zq = zq_ref[0]
    batch_term = (jnp.sum(zq * zq)
                  - 2.0 * (jnp.sum(rmax0) + jnp.sum(rmax1)) + zn_sq)

    @pl.when(b == 0)
    def _():
        loss_ref[...] = jnp.zeros((1, 1), jnp.float32)

    loss_ref[...] += jnp.full((1, 1), (_BETA / (_B * _HW * _C)),
                              jnp.float32) * batch_term


def kernel(z, embedding):
    zf = z.reshape(_B, _C, _HW)
    kio = jnp.arange(_K, dtype=jnp.float32)
    et_aug = jnp.concatenate(
        [embedding.T,
         jnp.floor(kio / 128.0)[None, :],
         jnp.mod(kio, 128.0)[None, :],
         jnp.ones((1, _K), jnp.float32)], axis=0).astype(jnp.bfloat16)
    ids3, zq3, loss = pl.pallas_call(
        _vq_body,
        grid=(_B,),
        in_specs=[
            pl.BlockSpec((1, _C, _HW), lambda b: (b, 0, 0)),
            pl.BlockSpec((_K, _C), lambda b: (0, 0)),
            pl.BlockSpec((_C + 3, _K), lambda b: (0, 0)),
        ],
        out_specs=[
            pl.BlockSpec((1, 1, _HW), lambda b: (b, 0, 0)),
            pl.BlockSpec((1, _C, _HW), lambda b: (b, 0, 0)),
            pl.BlockSpec((1, 1), lambda b: (0, 0)),
        ],
        out_shape=[
            jax.ShapeDtypeStruct((_B, 1, _HW), jnp.int32),
            jax.ShapeDtypeStruct((_B, _C, _HW), jnp.float32),
            jax.ShapeDtypeStruct((1, 1), jnp.float32),
        ],
        scratch_shapes=[
            pltpu.VMEM((_NCH, _KC, _HH), jnp.float32),
            pltpu.VMEM((_NCH, _KC, _HH), jnp.float32),
            pltpu.VMEM((_C, _HW), jnp.float32),
            pltpu.VMEM((_C + 3, _HW), jnp.float32),
        ],
    )(zf, embedding, et_aug)
    z_q_out = zq3.reshape(_B, _C, _H, _W)
    token_ids = ids3.reshape(_B, _H, _W)
    return (z_q_out, loss[0, 0], token_ids)


# speedup vs baseline: 1.5100x; 1.5100x over previous
import jax
import jax.numpy as jnp
from jax import lax
from jax.experimental import pallas as pl
from jax.experimental.pallas import tpu as pltpu

_K = 8192
_C = 64
_BETA = 0.25
_B, _H, _W = 8, 32, 32
_HW = _H * _W
_KC = 512
_NCH = _K // _KC
_UNROLL = 4


def _vq_body(z_ref, e_ref, et_ref, ids_ref, zq_ref, loss_ref, s_ref, acc_ref):
    b = pl.program_id(0)
    z = z_ref[0]
    nsq = jnp.sum(z * z, axis=0, keepdims=True)
    n = jnp.sqrt(nsq)
    zn = z / jnp.maximum(n, 1e-12)
    zn_sq = jnp.sum(zn * zn)

    def pass1(k, rmax):
        e_blk = e_ref[pl.ds(k * _KC, _KC), :]
        sim = lax.dot(e_blk, zn)
        s_ref[k] = sim
        return jnp.maximum(rmax, jnp.max(sim, axis=0, keepdims=True))

    rmax = lax.fori_loop(
        0, _NCH, pass1, jnp.full((1, _HW), -jnp.inf, dtype=jnp.float32),
        unroll=_UNROLL)

    acc_ref[...] = jnp.zeros((_C + 3, _HW), jnp.float32)

    def pass2(k, _):
        onehot = (s_ref[k] == rmax).astype(jnp.bfloat16)
        g_blk = et_ref[:, pl.ds(k * _KC, _KC)]
        acc_ref[...] += lax.dot(g_blk, onehot,
                                preferred_element_type=jnp.float32)
        return 0

    lax.fori_loop(0, _NCH, pass2, 0, unroll=_UNROLL)
    acc = acc_ref[...]
    count = acc[_C + 2:_C + 3]
    ids = (acc[_C:_C + 1] * 128.0 + acc[_C + 1:_C + 2]).astype(jnp.int32)
    ids_ref[0] = ids
    zq_ref[0] = acc[:_C]

    has_tie = jnp.any(count != 1.0)

    @pl.when(has_tie)
    def _():
        def find(k, ridx):
            iota = lax.broadcasted_iota(jnp.int32, (_KC, _HW), 0) + k * _KC
            bidx = jnp.min(jnp.where(s_ref[k] == rmax, iota, _K),
                           axis=0, keepdims=True)
            return jnp.minimum(ridx, bidx)

        ids_x = lax.fori_loop(0, _NCH, find,
                              jnp.full((1, _HW), _K, dtype=jnp.int32))
        ids_ref[0] = ids_x
        acc_ref[...] = jnp.zeros((_C + 3, _HW), jnp.float32)

        def rebuild(k, _):
            iota = lax.broadcasted_iota(jnp.int32, (_KC, _HW), 0) + k * _KC
            onehot = (iota == ids_x).astype(jnp.bfloat16)
            g_blk = et_ref[:, pl.ds(k * _KC, _KC)]
            acc_ref[...] += lax.dot(g_blk, onehot,
                                    preferred_element_type=jnp.float32)
            return 0

        lax.fori_loop(0, _NCH, rebuild, 0)
        zq_ref[0] = acc_ref[:_C]

    zq = zq_ref[0]
    batch_term = jnp.sum(zq * zq) - 2.0 * jnp.sum(rmax) + zn_sq

    @pl.when(b == 0)
    def _():
        loss_ref[...] = jnp.zeros((1, 1), jnp.float32)

    loss_ref[...] += jnp.full((1, 1), (_BETA / (_B * _HW * _C)),
                              jnp.float32) * batch_term


def kernel(z, embedding):
    zf = z.reshape(_B, _C, _HW)
    kio = jnp.arange(_K, dtype=jnp.float32)
    et_aug = jnp.concatenate(
        [embedding.T,
         jnp.floor(kio / 128.0)[None, :],
         jnp.mod(kio, 128.0)[None, :],
         jnp.ones((1, _K), jnp.float32)], axis=0).astype(jnp.bfloat16)
    ids3, zq3, loss = pl.pallas_call(
        _vq_body,
        grid=(_B,),
        in_specs=[
            pl.BlockSpec((1, _C, _HW), lambda b: (b, 0, 0)),
            pl.BlockSpec((_K, _C), lambda b: (0, 0)),
            pl.BlockSpec((_C + 3, _K), lambda b: (0, 0)),
        ],
        out_specs=[
            pl.BlockSpec((1, 1, _HW), lambda b: (b, 0, 0)),
            pl.BlockSpec((1, _C, _HW), lambda b: (b, 0, 0)),
            pl.BlockSpec((1, 1), lambda b: (0, 0)),
        ],
        out_shape=[
            jax.ShapeDtypeStruct((_B, 1, _HW), jnp.int32),
            jax.ShapeDtypeStruct((_B, _C, _HW), jnp.float32),
            jax.ShapeDtypeStruct((1, 1), jnp.float32),
        ],
        scratch_shapes=[
            pltpu.VMEM((_NCH, _KC, _HW), jnp.float32),
            pltpu.VMEM((_C + 3, _HW), jnp.float32),
        ],
    )(zf, embedding, et_aug)
    z_q_out = zq3.reshape(_B, _C, _H, _W)
    token_ids = ids3.reshape(_B, _H, _W)
    return (z_q_out, loss[0, 0], token_ids)


# speedup vs baseline: 1.5780x; 1.0451x over previous
import jax
import jax.numpy as jnp
from jax import lax
from jax.experimental import pallas as pl
from jax.experimental.pallas import tpu as pltpu

_K = 8192
_C = 64
_BETA = 0.25
_B, _H, _W = 8, 32, 32
_HW = _H * _W
_KC = 512
_NCH = _K // _KC
_UNROLL = 8


def _vq_body(z_ref, e_ref, et_ref, ids_ref, zq_ref, loss_ref, s_ref, acc_ref):
    b = pl.program_id(0)
    z = z_ref[0]
    nsq = jnp.sum(z * z, axis=0, keepdims=True)
    n = jnp.sqrt(nsq)
    zn = z / jnp.maximum(n, 1e-12)
    zn_sq = jnp.sum(zn * zn)

    def pass1(k, rmax):
        e_blk = e_ref[pl.ds(k * _KC, _KC), :]
        sim = lax.dot(e_blk, zn)
        s_ref[k] = sim
        return jnp.maximum(rmax, jnp.max(sim, axis=0, keepdims=True))

    rmax = lax.fori_loop(
        0, _NCH, pass1, jnp.full((1, _HW), -jnp.inf, dtype=jnp.float32),
        unroll=_UNROLL)

    acc_ref[...] = jnp.zeros((_C + 3, _HW), jnp.float32)

    def pass2(k, _):
        onehot = (s_ref[k] == rmax).astype(jnp.bfloat16)
        g_blk = et_ref[:, pl.ds(k * _KC, _KC)]
        acc_ref[...] += lax.dot(g_blk, onehot,
                                preferred_element_type=jnp.float32)
        return 0

    lax.fori_loop(0, _NCH, pass2, 0, unroll=_UNROLL)
    acc = acc_ref[...]
    count = acc[_C + 2:_C + 3]
    ids = (acc[_C:_C + 1] * 128.0 + acc[_C + 1:_C + 2]).astype(jnp.int32)
    ids_ref[0] = ids
    zq_ref[0] = acc[:_C]

    has_tie = jnp.any(count != 1.0)

    @pl.when(has_tie)
    def _():
        def find(k, ridx):
            iota = lax.broadcasted_iota(jnp.int32, (_KC, _HW), 0) + k * _KC
            bidx = jnp.min(jnp.where(s_ref[k] == rmax, iota, _K),
                           axis=0, keepdims=True)
            return jnp.minimum(ridx, bidx)

        ids_x = lax.fori_loop(0, _NCH, find,
                              jnp.full((1, _HW), _K, dtype=jnp.int32))
        ids_ref[0] = ids_x
        acc_ref[...] = jnp.zeros((_C + 3, _HW), jnp.float32)

        def rebuild(k, _):
            iota = lax.broadcasted_iota(jnp.int32, (_KC, _HW), 0) + k * _KC
            onehot = (iota == ids_x).astype(jnp.bfloat16)
            g_blk = et_ref[:, pl.ds(k * _KC, _KC)]
            acc_ref[...] += lax.dot(g_blk, onehot,
                                    preferred_element_type=jnp.float32)
            return 0

        lax.fori_loop(0, _NCH, rebuild, 0)
        zq_ref[0] = acc_ref[:_C]

    zq = zq_ref[0]
    batch_term = jnp.sum(zq * zq) - 2.0 * jnp.sum(rmax) + zn_sq

    @pl.when(b == 0)
    def _():
        loss_ref[...] = jnp.zeros((1, 1), jnp.float32)

    loss_ref[...] += jnp.full((1, 1), (_BETA / (_B * _HW * _C)),
                              jnp.float32) * batch_term


def kernel(z, embedding):
    zf = z.reshape(_B, _C, _HW)
    kio = jnp.arange(_K, dtype=jnp.float32)
    et_aug = jnp.concatenate(
        [embedding.T,
         jnp.floor(kio / 128.0)[None, :],
         jnp.mod(kio, 128.0)[None, :],
         jnp.ones((1, _K), jnp.float32)], axis=0).astype(jnp.bfloat16)
    ids3, zq3, loss = pl.pallas_call(
        _vq_body,
        grid=(_B,),
        in_specs=[
            pl.BlockSpec((1, _C, _HW), lambda b: (b, 0, 0)),
            pl.BlockSpec((_K, _C), lambda b: (0, 0)),
            pl.BlockSpec((_C + 3, _K), lambda b: (0, 0)),
        ],
        out_specs=[
            pl.BlockSpec((1, 1, _HW), lambda b: (b, 0, 0)),
            pl.BlockSpec((1, _C, _HW), lambda b: (b, 0, 0)),
            pl.BlockSpec((1, 1), lambda b: (0, 0)),
        ],
        out_shape=[
            jax.ShapeDtypeStruct((_B, 1, _HW), jnp.int32),
            jax.ShapeDtypeStruct((_B, _C, _HW), jnp.float32),
            jax.ShapeDtypeStruct((1, 1), jnp.float32),
        ],
        scratch_shapes=[
            pltpu.VMEM((_NCH, _KC, _HW), jnp.float32),
            pltpu.VMEM((_C + 3, _HW), jnp.float32),
        ],
    )(zf, embedding, et_aug)
    z_q_out = zq3.reshape(_B, _C, _H, _W)
    token_ids = ids3.reshape(_B, _H, _W)
    return (z_q_out, loss[0, 0], token_ids)


# speedup vs baseline: 1.6238x; 1.0290x over previous
import jax
import jax.numpy as jnp
from jax import lax
from jax.experimental import pallas as pl
from jax.experimental.pallas import tpu as pltpu

_K = 8192
_C = 64
_BETA = 0.25
_B, _H, _W = 8, 32, 32
_HW = _H * _W
_KC = 512
_NCH = _K // _KC
_UNROLL = 8


def _vq_body(z_ref, e_ref, et_ref, ids_ref, zq_ref, loss_ref, s_ref, acc_ref):
    b = pl.program_id(0)
    z = z_ref[0]
    nsq = jnp.sum(z * z, axis=0, keepdims=True)
    n = jnp.sqrt(nsq)
    zn = z / jnp.maximum(n, 1e-12)
    zn_sq = jnp.sum(zn * zn)
    znh = zn.astype(jnp.bfloat16)

    def pass1(k, rmax):
        e_blk = e_ref[pl.ds(k * _KC, _KC), :]
        sim = lax.dot(e_blk, znh,
                      preferred_element_type=jnp.float32)
        s_ref[k] = sim
        return jnp.maximum(rmax, jnp.max(sim, axis=0, keepdims=True))

    rmax = lax.fori_loop(
        0, _NCH, pass1, jnp.full((1, _HW), -jnp.inf, dtype=jnp.float32),
        unroll=_UNROLL)

    acc_ref[...] = jnp.zeros((_C + 3, _HW), jnp.float32)

    def pass2(k, _):
        onehot = (s_ref[k] == rmax).astype(jnp.bfloat16)
        g_blk = et_ref[:, pl.ds(k * _KC, _KC)]
        acc_ref[...] += lax.dot(g_blk, onehot,
                                preferred_element_type=jnp.float32)
        return 0

    lax.fori_loop(0, _NCH, pass2, 0, unroll=_UNROLL)
    acc = acc_ref[...]
    count = acc[_C + 2:_C + 3]
    ids = (acc[_C:_C + 1] * 128.0 + acc[_C + 1:_C + 2]).astype(jnp.int32)
    ids_ref[0] = ids
    zq_ref[0] = acc[:_C]

    has_tie = jnp.any(count != 1.0)

    @pl.when(has_tie)
    def _():
        def find(k, ridx):
            iota = lax.broadcasted_iota(jnp.int32, (_KC, _HW), 0) + k * _KC
            bidx = jnp.min(jnp.where(s_ref[k] == rmax, iota, _K),
                           axis=0, keepdims=True)
            return jnp.minimum(ridx, bidx)

        ids_x = lax.fori_loop(0, _NCH, find,
                              jnp.full((1, _HW), _K, dtype=jnp.int32))
        ids_ref[0] = ids_x
        acc_ref[...] = jnp.zeros((_C + 3, _HW), jnp.float32)

        def rebuild(k, _):
            iota = lax.broadcasted_iota(jnp.int32, (_KC, _HW), 0) + k * _KC
            onehot = (iota == ids_x).astype(jnp.bfloat16)
            g_blk = et_ref[:, pl.ds(k * _KC, _KC)]
            acc_ref[...] += lax.dot(g_blk, onehot,
                                    preferred_element_type=jnp.float32)
            return 0

        lax.fori_loop(0, _NCH, rebuild, 0)
        zq_ref[0] = acc_ref[:_C]

    zq = zq_ref[0]
    batch_term = jnp.sum(zq * zq) - 2.0 * jnp.sum(rmax) + zn_sq

    @pl.when(b == 0)
    def _():
        loss_ref[...] = jnp.zeros((1, 1), jnp.float32)

    loss_ref[...] += jnp.full((1, 1), (_BETA / (_B * _HW * _C)),
                              jnp.float32) * batch_term


def kernel(z, embedding):
    zf = z.reshape(_B, _C, _HW)
    kio = jnp.arange(_K, dtype=jnp.float32)
    et_aug = jnp.concatenate(
        [embedding.T,
         jnp.floor(kio / 128.0)[None, :],
         jnp.mod(kio, 128.0)[None, :],
         jnp.ones((1, _K), jnp.float32)], axis=0).astype(jnp.bfloat16)
    ids3, zq3, loss = pl.pallas_call(
        _vq_body,
        grid=(_B,),
        in_specs=[
            pl.BlockSpec((1, _C, _HW), lambda b: (b, 0, 0)),
            pl.BlockSpec((_K, _C), lambda b: (0, 0)),
            pl.BlockSpec((_C + 3, _K), lambda b: (0, 0)),
        ],
        out_specs=[
            pl.BlockSpec((1, 1, _HW), lambda b: (b, 0, 0)),
            pl.BlockSpec((1, _C, _HW), lambda b: (b, 0, 0)),
            pl.BlockSpec((1, 1), lambda b: (0, 0)),
        ],
        out_shape=[
            jax.ShapeDtypeStruct((_B, 1, _HW), jnp.int32),
            jax.ShapeDtypeStruct((_B, _C, _HW), jnp.float32),
            jax.ShapeDtypeStruct((1, 1), jnp.float32),
        ],
        scratch_shapes=[
            pltpu.VMEM((_NCH, _KC, _HW), jnp.float32),
            pltpu.VMEM((_C + 3, _HW), jnp.float32),
        ],
    )(zf, embedding.astype(jnp.bfloat16), et_aug)
    z_q_out = zq3.reshape(_B, _C, _H, _W)
    token_ids = ids3.reshape(_B, _H, _W)
    return (z_q_out, loss[0, 0], token_ids)
